# baseline (device time: 125535 ns/iter reference)
import os

import jax
import jax.numpy as jnp
from jax import lax
from jax.experimental import pallas as pl
from jax.experimental.pallas import tpu as pltpu

N_DEV = 8
N_KC = 2

_VARIANT = os.environ.get("KERNEL_VARIANT", "full")


def kernel(x, w_mat):
    m_per, k = x.shape
    _, n = w_mat.shape
    n_per = n // N_DEV
    k_tile = k // N_KC
    n_steps = N_DEV * N_KC

    x = x.astype(jnp.bfloat16)

    def body(
        x_ref, w_hbm, out_ref, w_stage, acc_ref, send_buf, w_sems, send_sems, recv_sems
    ):
        my = lax.axis_index("i")

        if _VARIANT != "nocomm":
            bar = pltpu.get_barrier_semaphore()
            for p in range(N_DEV):
                pl.semaphore_signal(
                    bar, inc=1, device_id=(p,), device_id_type=pl.DeviceIdType.MESH
                )
            pl.semaphore_wait(bar, N_DEV)

        def tgt_dev(blk):
            return jnp.where(blk < N_DEV - 1, lax.rem(my + 1 + blk, N_DEV), my)

        def w_dma(s):
            kc = s % N_KC
            return pltpu.make_async_copy(
                w_hbm.at[
                    pl.ds(kc * k_tile, k_tile),
                    pl.ds(tgt_dev(s // N_KC) * n_per, n_per),
                ],
                w_stage.at[s % 2],
                w_sems.at[s % 2],
            )

        def send_desc(sslot, blk):
            return pltpu.make_async_remote_copy(
                src_ref=send_buf.at[sslot],
                dst_ref=out_ref.at[pl.ds(my * m_per, m_per), :],
                send_sem=send_sems.at[sslot],
                recv_sem=recv_sems.at[blk],
                device_id=(tgt_dev(blk),),
                device_id_type=pl.DeviceIdType.MESH,
            )

        w_dma(jnp.int32(0)).start()

        def step(s, carry):
            blk = s // N_KC
            kc = s % N_KC
            w_dma(s).wait()

            @pl.when(s + 1 < n_steps)
            def _():
                w_dma(s + 1).start()

            if _VARIANT != "nocompute":
                partial = jnp.dot(
                    x_ref[:, pl.ds(kc * k_tile, k_tile)],
                    w_stage[s % 2].astype(jnp.bfloat16),
                    preferred_element_type=jnp.float32,
                )
            else:
                partial = jnp.full(
                    (m_per, n_per), w_stage[s % 2, 0, 0], dtype=jnp.float32
                )

            @pl.when(kc == 0)
            def _():
                acc_ref[...] = partial

            @pl.when(kc == N_KC - 1)
            def _():
                yb = (acc_ref[...] + partial).astype(jnp.bfloat16)

                @pl.when(blk == N_DEV - 1)
                def _():
                    out_ref[pl.ds(my * m_per, m_per), :] = yb

                @pl.when(blk < N_DEV - 1)
                def _():
                    send_buf[blk, :, :] = yb

                    if _VARIANT != "nocomm":
                        send_desc(blk, blk).start()

            return carry

        lax.fori_loop(0, n_steps, step, 0)

        if _VARIANT != "nocomm":
            for blk in range(N_DEV - 1):
                send_desc(blk, jnp.int32(blk)).wait_send()

            for blk in range(N_DEV - 1):
                recv = pltpu.make_async_remote_copy(
                    src_ref=send_buf.at[0],
                    dst_ref=out_ref.at[pl.ds(0, m_per), :],
                    send_sem=send_sems.at[0],
                    recv_sem=recv_sems.at[blk],
                    device_id=(my,),
                    device_id_type=pl.DeviceIdType.MESH,
                )
                recv.wait_recv()

    out_shape = jax.ShapeDtypeStruct((N_DEV * m_per, n_per), jnp.bfloat16)
    return pl.pallas_call(
        body,
        out_shape=out_shape,
        in_specs=[
            pl.BlockSpec(memory_space=pltpu.VMEM),
            pl.BlockSpec(memory_space=pl.ANY),
        ],
        out_specs=pl.BlockSpec(memory_space=pltpu.VMEM),
        scratch_shapes=[
            pltpu.VMEM((2, k_tile, n_per), jnp.float32),
            pltpu.VMEM((m_per, n_per), jnp.float32),
            pltpu.VMEM((N_DEV - 1, m_per, n_per), jnp.bfloat16),
            pltpu.SemaphoreType.DMA((2,)),
            pltpu.SemaphoreType.DMA((N_DEV - 1,)),
            pltpu.SemaphoreType.DMA((N_DEV - 1,)),
        ],
        compiler_params=pltpu.CompilerParams(
            collective_id=None if _VARIANT == "nocomm" else 0,
            vmem_limit_bytes=64 * 1024 * 1024,
        ),
    )(x, w_mat)


# device time: 115606 ns/iter; 1.0859x vs baseline; 1.0859x over previous
import os

import jax
import jax.numpy as jnp
from jax import lax
from jax.experimental import pallas as pl
from jax.experimental.pallas import tpu as pltpu

N_DEV = 8
N_KC = 2

_VARIANT = os.environ.get("KERNEL_VARIANT", "full")


def kernel(x, w_mat):
    m_per, k = x.shape
    _, n = w_mat.shape
    n_per = n // N_DEV
    k_tile = k // N_KC
    n_steps = N_DEV * N_KC
    m_chunk = 32
    n_xs = 4
    n_xc = m_per // m_chunk
    N_SS = 7

    def body(
        x_hbm,
        w_hbm,
        out_ref,
        x_bf,
        x_stage,
        w_stage,
        acc_ref,
        send_buf,
        x_sems,
        w_sems,
        send_sems,
        recv_sems,
    ):
        my = lax.axis_index("i")

        if _VARIANT != "nocomm":
            bar = pltpu.get_barrier_semaphore()
            for p in range(N_DEV):
                pl.semaphore_signal(
                    bar, inc=1, device_id=(p,), device_id_type=pl.DeviceIdType.MESH
                )
            pl.semaphore_wait(bar, N_DEV)

        def tgt_dev(blk):
            return jnp.where(blk < N_DEV - 1, lax.rem(my + 1 + blk, N_DEV), my)

        def w_dma(s):
            kc = s % N_KC
            return pltpu.make_async_copy(
                w_hbm.at[
                    pl.ds(kc * k_tile, k_tile),
                    pl.ds(tgt_dev(s // N_KC) * n_per, n_per),
                ],
                w_stage.at[s % 2],
                w_sems.at[s % 2],
            )

        def send_desc(sslot, blk):
            return pltpu.make_async_remote_copy(
                src_ref=send_buf.at[sslot],
                dst_ref=out_ref.at[pl.ds(my * m_per, m_per), :],
                send_sem=send_sems.at[sslot],
                recv_sem=recv_sems.at[blk],
                device_id=(tgt_dev(blk),),
                device_id_type=pl.DeviceIdType.MESH,
            )

        def x_dma(c):
            return pltpu.make_async_copy(
                x_hbm.at[pl.ds(c * m_chunk, m_chunk), :],
                x_stage.at[c % n_xs],
                x_sems.at[c % n_xs],
            )

        if _VARIANT != "nowdma":
            w_dma(jnp.int32(0)).start()
            w_dma(jnp.int32(1)).start()

        for c in range(n_xs):
            x_dma(c).start()
        for c in range(n_xc):
            x_dma(c).wait()
            if c + n_xs < n_xc:
                x_dma(c + n_xs).start()
            x_bf[pl.ds(c * m_chunk, m_chunk), :] = x_stage[c % n_xs].astype(
                jnp.bfloat16
            )

        def step(s, carry):
            blk = s // N_KC
            kc = s % N_KC
            if _VARIANT != "nowdma":
                w_dma(s).wait()

            if _VARIANT != "nocompute":
                partial = jnp.dot(
                    x_bf[:, pl.ds(kc * k_tile, k_tile)],
                    w_stage[s % 2].astype(jnp.bfloat16),
                    preferred_element_type=jnp.float32,
                )
            else:
                partial = jnp.full(
                    (m_per, n_per), w_stage[s % 2, 0, 0], dtype=jnp.float32
                )

            if _VARIANT != "nowdma":
                @pl.when(s + 2 < n_steps)
                def _():
                    w_dma(s + 2).start()

            @pl.when(kc == 0)
            def _():
                acc_ref[...] = partial

            @pl.when(kc == N_KC - 1)
            def _():
                yb = (acc_ref[...] + partial).astype(jnp.bfloat16)

                @pl.when(blk == N_DEV - 1)
                def _():
                    out_ref[pl.ds(my * m_per, m_per), :] = yb

                @pl.when(blk < N_DEV - 1)
                def _():
                    if _VARIANT != "nocomm":
                        @pl.when(blk >= N_SS)
                        def _():
                            send_desc(blk % N_SS, blk - N_SS).wait_send()

                    send_buf[blk % N_SS, :, :] = yb

                    if _VARIANT != "nocomm":
                        send_desc(blk % N_SS, blk).start()

            return carry

        lax.fori_loop(0, n_steps, step, 0)

        if _VARIANT != "nocomm":
            for blk in range(max(0, N_DEV - 1 - N_SS), N_DEV - 1):
                send_desc(blk % N_SS, jnp.int32(blk)).wait_send()

            for blk in range(N_DEV - 1):
                recv = pltpu.make_async_remote_copy(
                    src_ref=send_buf.at[0],
                    dst_ref=out_ref.at[pl.ds(0, m_per), :],
                    send_sem=send_sems.at[0],
                    recv_sem=recv_sems.at[blk],
                    device_id=(my,),
                    device_id_type=pl.DeviceIdType.MESH,
                )
                recv.wait_recv()

    out_shape = jax.ShapeDtypeStruct((N_DEV * m_per, n_per), jnp.bfloat16)
    return pl.pallas_call(
        body,
        out_shape=out_shape,
        in_specs=[
            pl.BlockSpec(memory_space=pl.ANY),
            pl.BlockSpec(memory_space=pl.ANY),
        ],
        out_specs=pl.BlockSpec(memory_space=pltpu.VMEM),
        scratch_shapes=[
            pltpu.VMEM((m_per, k), jnp.bfloat16),
            pltpu.VMEM((n_xs, m_chunk, k), jnp.float32),
            pltpu.VMEM((2, k_tile, n_per), jnp.float32),
            pltpu.VMEM((m_per, n_per), jnp.float32),
            pltpu.VMEM((N_SS, m_per, n_per), jnp.bfloat16),
            pltpu.SemaphoreType.DMA((n_xs,)),
            pltpu.SemaphoreType.DMA((2,)),
            pltpu.SemaphoreType.DMA((N_SS,)),
            pltpu.SemaphoreType.DMA((N_DEV - 1,)),
        ],
        compiler_params=pltpu.CompilerParams(
            collective_id=None if _VARIANT == "nocomm" else 0,
            vmem_limit_bytes=64 * 1024 * 1024,
        ),
    )(x, w_mat)
